# speedup vs baseline: 8.5996x; 8.5996x over previous
import functools

import jax
import jax.numpy as jnp
from jax import lax
from jax.experimental import pallas as pl
from jax.experimental.pallas import tpu as pltpu
from jax.experimental.pallas import tpu_sc as plsc

N = 10000
E = 320000
D = 128
NPAD = 10240
NC, NS = 2, 16
NW = NC * NS
CH = 128
K = (E + NW * CH - 1) // (NW * CH)
EPAD = NW * CH * K
RPT = NPAD // NS
BR = 1024

_sc_mesh = plsc.VectorSubcoreMesh(core_axis_name="c", subcore_axis_name="s")


def _deg_body(dst3, zeros16, ones16, out, dstv, onesv, tab):
    cid = lax.axis_index("c")
    sid = lax.axis_index("s")
    wid = sid * NC + cid
    sl = pl.ds(sid * RPT, RPT)
    pltpu.sync_copy(zeros16.at[sl], tab.at[sl])
    pltpu.sync_copy(ones16, onesv)
    pltpu.sync_copy(dst3.at[wid], dstv)
    plsc.subcore_barrier()

    def chunk(c, carry):
        pltpu.sync_copy(onesv, tab.at[dstv.at[c]], add=True)
        return carry

    lax.fori_loop(0, K, chunk, 0)
    plsc.subcore_barrier()
    pltpu.sync_copy(tab.at[sl], out.at[cid, sl])


_deg_kernel = pl.kernel(
    _deg_body,
    out_type=jax.ShapeDtypeStruct((NC, NPAD, 16), jnp.float32),
    mesh=_sc_mesh,
    scratch_types=[
        pltpu.VMEM((K, CH), jnp.int32),
        pltpu.VMEM((CH, 16), jnp.float32),
        pltpu.VMEM_SHARED((NPAD, 16), jnp.float32),
    ],
)


def _scat_body(g, src3, dst3, zeros, out, srcv, dstv, rows, acc):
    cid = lax.axis_index("c")
    sid = lax.axis_index("s")
    wid = sid * NC + cid
    sl = pl.ds(sid * RPT, RPT)
    pltpu.sync_copy(zeros.at[sl], acc.at[sl])
    pltpu.sync_copy(src3.at[wid], srcv)
    pltpu.sync_copy(dst3.at[wid], dstv)
    plsc.subcore_barrier()

    def chunk(c, carry):
        pltpu.sync_copy(g.at[srcv.at[c]], rows)
        pltpu.sync_copy(rows, acc.at[dstv.at[c]], add=True)
        return carry

    lax.fori_loop(0, K, chunk, 0)
    plsc.subcore_barrier()
    pltpu.sync_copy(acc.at[sl], out.at[cid, sl])


_scat_kernel = pl.kernel(
    _scat_body,
    out_type=jax.ShapeDtypeStruct((NC, NPAD, D), jnp.float32),
    mesh=_sc_mesh,
    scratch_types=[
        pltpu.VMEM((K, CH), jnp.int32),
        pltpu.VMEM((K, CH), jnp.int32),
        pltpu.VMEM((CH, D), jnp.float32),
        pltpu.VMEM_SHARED((NPAD, D), jnp.float32),
    ],
)


def _t0_body(x_ref, d0_ref, d1_ref, w_ref, dinv_ref, g_ref):
    deg = d0_ref[...] + d1_ref[...] + 1.0
    dinv = lax.rsqrt(jnp.maximum(deg, 1.0))
    dinv_ref[...] = dinv
    g_ref[...] = jnp.dot(x_ref[...], w_ref[...],
                         preferred_element_type=jnp.float32) * dinv


_row = pl.BlockSpec((BR, D), lambda i: (i, 0))
_col1 = pl.BlockSpec((BR, 1), lambda i: (i, 0))
_full_w = pl.BlockSpec((D, D), lambda i: (0, 0))
_full_b = pl.BlockSpec((1, D), lambda i: (0, 0))

_t0_kernel = pl.pallas_call(
    _t0_body,
    grid=(NPAD // BR,),
    in_specs=[_row, _col1, _col1, _full_w],
    out_specs=[_col1, _row],
    out_shape=[
        jax.ShapeDtypeStruct((NPAD, 1), jnp.float32),
        jax.ShapeDtypeStruct((NPAD, D), jnp.float32),
    ],
)


def _tmid_body(p0_ref, p1_ref, g_ref, dinv_ref, b_ref, w_ref, out_ref):
    dinv = dinv_ref[...]
    t = dinv * (p0_ref[...] + p1_ref[...] + g_ref[...]) + b_ref[...]
    t = jnp.maximum(t, 0.0)
    out_ref[...] = jnp.dot(t, w_ref[...],
                           preferred_element_type=jnp.float32) * dinv


_tmid_kernel = pl.pallas_call(
    _tmid_body,
    grid=(NPAD // BR,),
    in_specs=[_row, _row, _row, _col1, _full_b, _full_w],
    out_specs=_row,
    out_shape=jax.ShapeDtypeStruct((NPAD, D), jnp.float32),
)


def _tfin_body(p0_ref, p1_ref, g_ref, dinv_ref, b_ref, out_ref):
    out_ref[...] = (dinv_ref[...] * (p0_ref[...] + p1_ref[...] + g_ref[...])
                    + b_ref[...])


_tfin_kernel = pl.pallas_call(
    _tfin_body,
    grid=(NPAD // BR,),
    in_specs=[_row, _row, _row, _col1, _full_b],
    out_specs=_row,
    out_shape=jax.ShapeDtypeStruct((NPAD, D), jnp.float32),
)


def kernel(x, edge_index, W0, b0, W1, b1, W2, b2, W3, b3, W4, b4, W5, b5):
    Ws = [W0, W1, W2, W3, W4, W5]
    bs = [b0, b1, b2, b3, b4, b5]

    src = edge_index[0]
    dst = edge_index[1]
    pad = EPAD - E
    src3 = jnp.concatenate(
        [src, jnp.zeros((pad,), jnp.int32)]).reshape(NW, K, CH)
    dst3 = jnp.concatenate(
        [dst, jnp.full((pad,), N, jnp.int32)]).reshape(NW, K, CH)

    zeros = jnp.zeros((NPAD, D), jnp.float32)
    zeros16 = jnp.zeros((NPAD, 16), jnp.float32)
    ones16 = jnp.ones((CH, 16), jnp.float32)
    x_pad = jnp.concatenate([x, jnp.zeros((NPAD - N, D), jnp.float32)])

    degp = _deg_kernel(dst3, zeros16, ones16)
    d0 = degp[0, :, 0:1]
    d1 = degp[1, :, 0:1]

    dinv, g = _t0_kernel(x_pad, d0, d1, Ws[0])
    for i in range(1, 6):
        part = _scat_kernel(g, src3, dst3, zeros)
        g = _tmid_kernel(part[0], part[1], g, dinv,
                         bs[i - 1].reshape(1, D), Ws[i])
    part = _scat_kernel(g, src3, dst3, zeros)
    out = _tfin_kernel(part[0], part[1], g, dinv, bs[5].reshape(1, D))
    return out[:N]
